# speedup vs baseline: 1.8229x; 1.8229x over previous
import jax
import jax.numpy as jnp
import numpy as np
from jax.experimental import pallas as pl

N = 10000
E = 160000
D = 128
H = 8
DH = 16
EPS = 1e-8

_BLK = 1000
_NBLK = N // _BLK


def _proj_body(h_ref, qw_ref, qb_ref, kw_ref, kb_ref, vw_ref, vb_ref,
               bd0_ref, bd1_ref, qs_ref, kt0_ref, kt1_ref, v_ref):
    hb = h_ref[...]
    k = hb @ kw_ref[...] + kb_ref[...]
    qs_ref[...] = hb @ qw_ref[...] + qb_ref[...]
    kt0_ref[...] = k @ bd0_ref[...]
    kt1_ref[...] = k @ bd1_ref[...]
    v_ref[...] = hb @ vw_ref[...] + vb_ref[...]


def _out_body(hmsg_ref, h_ref, ow_ref, ob_ref, lns_ref, lnb_ref, o_ref):
    t = jax.nn.gelu(hmsg_ref[...]) @ ow_ref[...] + ob_ref[...]
    x = t + h_ref[...]
    mu = jnp.mean(x, axis=-1, keepdims=True)
    var = jnp.mean((x - mu) ** 2, axis=-1, keepdims=True)
    o_ref[...] = (x - mu) / jnp.sqrt(var + 1e-5) * lns_ref[...] + lnb_ref[...]


def kernel(h, src_idx0, dst_idx0, src_idx1, dst_idx1, Qw, Qb, Kw, Kb, Vw, Vb,
           edge_W, gate_logits, Ow, Ob, ln_s, ln_b):
    scale = 1.0 / np.sqrt(DH)
    bd = jnp.zeros((2, H, DH, H, DH), jnp.float32)
    bd = bd.at[:, jnp.arange(H), :, jnp.arange(H), :].set(
        jnp.transpose(edge_W, (1, 0, 2, 3))).reshape(2, D, D)

    row = lambda b: b.reshape(1, D)
    wspec = pl.BlockSpec((D, D), lambda i: (0, 0))
    bspec = pl.BlockSpec((1, D), lambda i: (0, 0))
    nspec = pl.BlockSpec((_BLK, D), lambda i: (i, 0))
    nshape = jax.ShapeDtypeStruct((N, D), jnp.float32)

    qs, kt0, kt1, v = pl.pallas_call(
        _proj_body,
        grid=(_NBLK,),
        in_specs=[nspec, wspec, bspec, wspec, bspec, wspec, bspec, wspec, wspec],
        out_specs=[nspec, nspec, nspec, nspec],
        out_shape=[nshape, nshape, nshape, nshape],
    )(h, Qw * scale, row(Qb) * scale, Kw, row(Kb), Vw, row(Vb), bd[0], bd[1])

    h_msg = jnp.zeros((N, D), jnp.float32)
    for r, (si, di) in enumerate(((src_idx0, dst_idx0), (src_idx1, dst_idx1))):
        kt = (kt0, kt1)[r]
        score = jnp.sum((qs[di] * kt[si]).reshape(E, H, DH), axis=-1)
        ex = jnp.exp(score)
        dn = jax.ops.segment_sum(ex, di, num_segments=N)
        attn = ex / jnp.maximum(dn[di], EPS)
        gate = jax.nn.sigmoid(gate_logits[r])
        msg = (attn[..., None] * v[si].reshape(E, H, DH) * gate).reshape(E, D)
        h_msg = h_msg + jax.ops.segment_sum(msg, di, num_segments=N)

    return pl.pallas_call(
        _out_body,
        grid=(_NBLK,),
        in_specs=[nspec, nspec, wspec, bspec, bspec, bspec],
        out_specs=nspec,
        out_shape=nshape,
    )(h_msg, h, Ow, row(Ob), row(ln_s), row(ln_b))
